# Initial kernel scaffold; baseline (speedup 1.0000x reference)
import jax
import jax.numpy as jnp
from jax.experimental import pallas as pl


def kernel(idx, W, B):
    raise NotImplementedError("write your pallas kernel here")


# speedup vs baseline: 60.6045x; 60.6045x over previous
import functools

import jax
import jax.numpy as jnp
from jax import lax
from jax.experimental import pallas as pl
from jax.experimental.pallas import tpu as pltpu
from jax.experimental.pallas import tpu_sc as plsc


def _hist_sc(idx, k):
    bsz, n_desc = idx.shape
    info = plsc.get_sparse_core_info()
    nc, ns, L = info.num_cores, info.num_subcores, info.num_lanes
    nw = nc * ns
    imgs_per_w = bsz // nw

    mesh = plsc.VectorSubcoreMesh(core_axis_name="c", subcore_axis_name="s")

    @functools.partial(
        pl.kernel,
        mesh=mesh,
        out_type=jax.ShapeDtypeStruct((bsz, k), jnp.float32),
        compiler_params=pltpu.CompilerParams(needs_layout_passes=False),
        scratch_types=[
            pltpu.VMEM((n_desc,), jnp.int32),
            pltpu.VMEM((k,), jnp.float32),
        ],
    )
    def hist_kernel(idx_hbm, out_hbm, idx_v, hist_v):
        wid = lax.axis_index("s") * nc + lax.axis_index("c")
        base = wid * imgs_per_w
        ones = jnp.full((L,), 1.0, jnp.float32)
        zeros = jnp.zeros((L,), jnp.float32)

        def img_body(i, carry):
            b = base + i
            pltpu.sync_copy(idx_hbm.at[b], idx_v)

            def zero_body(j, c):
                hist_v[pl.ds(j * L, L)] = zeros
                return c

            lax.fori_loop(0, k // L, zero_body, 0)

            def scat_body(d, c):
                v = idx_v[pl.ds(d * L, L)]
                plsc.addupdate_scatter(hist_v, [v], ones)
                return c

            lax.fori_loop(0, n_desc // L, scat_body, 0)
            pltpu.sync_copy(hist_v, out_hbm.at[b])
            return carry

        lax.fori_loop(0, imgs_per_w, img_body, 0)

    return hist_kernel(idx)


def _svm_tc(hist, W, B2):
    bsz, k = hist.shape
    ncls = W.shape[0]
    blk = 256

    def body(h_ref, w_ref, b_ref, o_ref):
        h = h_ref[...]
        ssq = jnp.sum(h * h, axis=1, keepdims=True)
        inv = 1.0 / (jnp.sqrt(ssq) + 1e-6)
        acc = lax.dot_general(h, w_ref[...], (((1,), (1,)), ((), ())),
                              preferred_element_type=jnp.float32)
        o_ref[...] = acc * inv + b_ref[...]

    return pl.pallas_call(
        body,
        grid=(bsz // blk,),
        in_specs=[
            pl.BlockSpec((blk, k), lambda i: (i, 0)),
            pl.BlockSpec((ncls, k), lambda i: (0, 0)),
            pl.BlockSpec((1, ncls), lambda i: (0, 0)),
        ],
        out_specs=pl.BlockSpec((blk, ncls), lambda i: (i, 0)),
        out_shape=jax.ShapeDtypeStruct((bsz, ncls), jnp.float32),
    )(hist, W, B2)


def kernel(idx, W, B):
    k = W.shape[1]
    hist = _hist_sc(idx, k)
    return _svm_tc(hist, W, B.reshape(1, -1))


# speedup vs baseline: 123.4789x; 2.0375x over previous
import functools

import jax
import jax.numpy as jnp
from jax import lax
from jax.experimental import pallas as pl
from jax.experimental.pallas import tpu as pltpu
from jax.experimental.pallas import tpu_sc as plsc


def _hist_sc(idx, k):
    bsz, n_desc = idx.shape
    info = plsc.get_sparse_core_info()
    nc, ns, L = info.num_cores, info.num_subcores, info.num_lanes
    nw = nc * ns
    imgs_per_w = bsz // nw
    NB = 16
    nbatches = imgs_per_w // NB

    mesh = plsc.VectorSubcoreMesh(core_axis_name="c", subcore_axis_name="s")

    @functools.partial(
        pl.kernel,
        mesh=mesh,
        out_type=jax.ShapeDtypeStruct((bsz * k,), jnp.float32),
        compiler_params=pltpu.CompilerParams(needs_layout_passes=False),
        scratch_types=[
            pltpu.VMEM((NB * n_desc,), jnp.int32),
            pltpu.VMEM((NB * n_desc,), jnp.int32),
            pltpu.VMEM((NB * k,), jnp.float32),
            pltpu.VMEM((NB * k,), jnp.float32),
            pltpu.SemaphoreType.DMA,
            pltpu.SemaphoreType.DMA,
            pltpu.SemaphoreType.DMA,
            pltpu.SemaphoreType.DMA,
        ],
    )
    def hist_kernel(idx_hbm, out_hbm, idx_v0, idx_v1, hist_v0, hist_v1,
                    si0, si1, so0, so1):
        wid = lax.axis_index("s") * nc + lax.axis_index("c")
        img0 = wid * imgs_per_w
        ones = jnp.full((L,), 1.0, jnp.float32)
        zeros = jnp.zeros((L,), jnp.float32)
        idx_bufs = [idx_v0, idx_v1]
        hist_bufs = [hist_v0, hist_v1]
        in_sems = [si0, si1]
        out_sems = [so0, so1]
        in_handles = [None, None]
        out_handles = [None, None]

        def start_in(t):
            s = t % 2
            src = idx_hbm.at[pl.ds((img0 + t * NB) * n_desc, NB * n_desc)]
            in_handles[s] = pltpu.async_copy(src, idx_bufs[s], in_sems[s])

        def zero_hist(s):
            hb = hist_bufs[s]

            @plsc.parallel_loop(0, NB * k // L, unroll=8)
            def _(i, _hb=hb):
                _hb[pl.ds(i * L, L)] = zeros

        start_in(0)
        start_in(1)
        zero_hist(0)
        zero_hist(1)

        for t in range(nbatches):
            s = t % 2
            ib, hb = idx_bufs[s], hist_bufs[s]
            in_handles[s].wait()
            if t >= 2:
                out_handles[s].wait()
                zero_hist(s)
            for j in range(NB):
                @plsc.parallel_loop(0, n_desc // L, unroll=8)
                def _(i, _jb=j * n_desc, _jo=j * k, _ib=ib, _hb=hb):
                    v = _ib[pl.ds(_jb + i * L, L)]
                    plsc.addupdate_scatter(_hb, [v + _jo], ones)

            dst = out_hbm.at[pl.ds((img0 + t * NB) * k, NB * k)]
            out_handles[s] = pltpu.async_copy(hb, dst, out_sems[s])
            if t + 2 < nbatches:
                start_in(t + 2)

        out_handles[(nbatches - 2) % 2].wait()
        out_handles[(nbatches - 1) % 2].wait()

    return hist_kernel(idx.reshape(-1)).reshape(bsz, k)


def _svm_tc(hist, W, B2):
    bsz, k = hist.shape
    ncls = W.shape[0]
    blk = 256

    def body(h_ref, w_ref, b_ref, o_ref):
        h = h_ref[...]
        ssq = jnp.sum(h * h, axis=1, keepdims=True)
        inv = 1.0 / (jnp.sqrt(ssq) + 1e-6)
        acc = lax.dot_general(h, w_ref[...], (((1,), (1,)), ((), ())),
                              preferred_element_type=jnp.float32)
        o_ref[...] = acc * inv + b_ref[...]

    return pl.pallas_call(
        body,
        grid=(bsz // blk,),
        in_specs=[
            pl.BlockSpec((blk, k), lambda i: (i, 0)),
            pl.BlockSpec((ncls, k), lambda i: (0, 0)),
            pl.BlockSpec((1, ncls), lambda i: (0, 0)),
        ],
        out_specs=pl.BlockSpec((blk, ncls), lambda i: (i, 0)),
        out_shape=jax.ShapeDtypeStruct((bsz, ncls), jnp.float32),
    )(hist, W, B2)


def kernel(idx, W, B):
    k = W.shape[1]
    hist = _hist_sc(idx, k)
    return _svm_tc(hist, W, B.reshape(1, -1))


# speedup vs baseline: 183.0988x; 1.4828x over previous
import functools

import jax
import jax.numpy as jnp
from jax import lax
from jax.experimental import pallas as pl
from jax.experimental.pallas import tpu as pltpu
from jax.experimental.pallas import tpu_sc as plsc


def _hist_sc(idx, k):
    bsz, n_desc = idx.shape
    info = plsc.get_sparse_core_info()
    nc, ns, L = info.num_cores, info.num_subcores, info.num_lanes
    nw = nc * ns
    imgs_per_w = bsz // nw
    NB = 16
    nbatches = imgs_per_w // NB

    mesh = plsc.VectorSubcoreMesh(core_axis_name="c", subcore_axis_name="s")

    @functools.partial(
        pl.kernel,
        mesh=mesh,
        out_type=jax.ShapeDtypeStruct((bsz, k), jnp.float32),
        compiler_params=pltpu.CompilerParams(needs_layout_passes=False),
        scratch_types=[
            pltpu.VMEM((NB, n_desc), jnp.int32),
            pltpu.VMEM((NB, n_desc), jnp.int32),
            pltpu.VMEM((NB, k), jnp.float32),
            pltpu.VMEM((NB, k), jnp.float32),
            pltpu.SemaphoreType.DMA,
            pltpu.SemaphoreType.DMA,
            pltpu.SemaphoreType.DMA,
            pltpu.SemaphoreType.DMA,
        ],
    )
    def hist_kernel(idx_hbm, out_hbm, idx_v0, idx_v1, hist_v0, hist_v1,
                    si0, si1, so0, so1):
        wid = lax.axis_index("s") * nc + lax.axis_index("c")
        img0 = wid * imgs_per_w
        ones = jnp.full((L,), 1.0, jnp.float32)
        zeros = jnp.zeros((L,), jnp.float32)
        idx_bufs = [idx_v0, idx_v1]
        hist_bufs = [hist_v0, hist_v1]
        in_sems = [si0, si1]
        out_sems = [so0, so1]
        in_handles = [None, None]
        out_handles = [None, None]
        rows = [jnp.full((L,), j, jnp.int32) for j in range(NB)]

        def start_in(t):
            s = t % 2
            src = idx_hbm.at[pl.ds(img0 + t * NB, NB)]
            in_handles[s] = pltpu.async_copy(src, idx_bufs[s], in_sems[s])

        def zero_hist(s):
            hb = hist_bufs[s]
            for j in range(NB):
                @plsc.parallel_loop(0, k // L, unroll=8)
                def _(i, _hb=hb, _j=j):
                    _hb[_j, pl.ds(i * L, L)] = zeros

        start_in(0)
        start_in(1)
        zero_hist(0)
        zero_hist(1)

        for t in range(nbatches):
            s = t % 2
            ib, hb = idx_bufs[s], hist_bufs[s]
            in_handles[s].wait()
            if t >= 2:
                out_handles[s].wait()
                zero_hist(s)
            for j in range(NB):
                @plsc.parallel_loop(0, n_desc // L, unroll=8)
                def _(i, _j=j, _ib=ib, _hb=hb):
                    v = _ib[_j, pl.ds(i * L, L)]
                    plsc.addupdate_scatter(_hb, [rows[_j], v], ones)

            dst = out_hbm.at[pl.ds(img0 + t * NB, NB)]
            out_handles[s] = pltpu.async_copy(hb, dst, out_sems[s])
            if t + 2 < nbatches:
                start_in(t + 2)

        out_handles[(nbatches - 2) % 2].wait()
        out_handles[(nbatches - 1) % 2].wait()

    return hist_kernel(idx)


def _svm_tc(hist, W, B2):
    bsz, k = hist.shape
    ncls = W.shape[0]
    blk = 256

    def body(h_ref, w_ref, b_ref, o_ref):
        h = h_ref[...]
        ssq = jnp.sum(h * h, axis=1, keepdims=True)
        inv = 1.0 / (jnp.sqrt(ssq) + 1e-6)
        acc = lax.dot_general(h, w_ref[...], (((1,), (1,)), ((), ())),
                              preferred_element_type=jnp.float32)
        o_ref[...] = acc * inv + b_ref[...]

    return pl.pallas_call(
        body,
        grid=(bsz // blk,),
        in_specs=[
            pl.BlockSpec((blk, k), lambda i: (i, 0)),
            pl.BlockSpec((ncls, k), lambda i: (0, 0)),
            pl.BlockSpec((1, ncls), lambda i: (0, 0)),
        ],
        out_specs=pl.BlockSpec((blk, ncls), lambda i: (i, 0)),
        out_shape=jax.ShapeDtypeStruct((bsz, ncls), jnp.float32),
    )(hist, W, B2)


def kernel(idx, W, B):
    k = W.shape[1]
    hist = _hist_sc(idx, k)
    return _svm_tc(hist, W, B.reshape(1, -1))
